# baseline (speedup 1.0000x reference)
import functools

import jax
import jax.numpy as jnp
from jax import lax
from jax.experimental import pallas as pl
from jax.experimental.pallas import tpu as pltpu
from jax.experimental.pallas import tpu_sc as plsc

ZMAXPAD = 88
DIM = 64
N_ATOMS = 1048576

NC = 2
NS = 16
NW = NC * NS
B_PER_W = N_ATOMS // NW
CHUNK = 512
N_CHUNKS = B_PER_W // CHUNK
L = 16


def kernel(species, table):
    mesh = plsc.VectorSubcoreMesh(core_axis_name="c", subcore_axis_name="s")

    @functools.partial(
        pl.kernel,
        mesh=mesh,
        compiler_params=pltpu.CompilerParams(use_tc_tiling_on_sc=False,
                                             needs_layout_passes=False),
        out_type=jax.ShapeDtypeStruct((NW, N_CHUNKS, CHUNK * DIM),
                                      jnp.float32),
        scratch_types=[
            pltpu.VMEM((ZMAXPAD * DIM,), jnp.float32),
            pltpu.VMEM((B_PER_W,), jnp.int32),
            pltpu.VMEM((CHUNK * DIM,), jnp.float32),
            pltpu.VMEM((CHUNK * DIM,), jnp.float32),
            pltpu.SemaphoreType.DMA,
            pltpu.SemaphoreType.DMA,
        ],
    )
    def sc_gather(species_hbm, table_hbm, out_hbm, table_v, idx_v,
                  rows0, rows1, wsem0, wsem1):
        wid = lax.axis_index("s") * NC + lax.axis_index("c")
        pltpu.sync_copy(table_hbm, table_v)
        pltpu.sync_copy(species_hbm.at[wid], idx_v)

        lanes = lax.iota(jnp.int32, L)
        scat_lanes = lanes * DIM

        def fill(j, buf):
            base = j * CHUNK

            @plsc.parallel_loop(0, CHUNK // L, unroll=2)
            def _group(g):
                sp = idx_v[pl.ds(base + g * L, L)]
                gather_base = sp * DIM
                scatter_base = g * (L * DIM) + scat_lanes
                for c in range(DIM):
                    v = plsc.load_gather(table_v, [gather_base + c])
                    plsc.store_scatter(buf, [scatter_base + c], v)

        def body(jj, _):
            j0 = 2 * jj
            j1 = j0 + 1

            @pl.when(jj > 0)
            def _drain0():
                pltpu.make_async_copy(rows0, out_hbm.at[wid, j0], wsem0).wait()

            fill(j0, rows0)
            pltpu.async_copy(rows0, out_hbm.at[wid, j0], wsem0)

            @pl.when(jj > 0)
            def _drain1():
                pltpu.make_async_copy(rows1, out_hbm.at[wid, j1], wsem1).wait()

            fill(j1, rows1)
            pltpu.async_copy(rows1, out_hbm.at[wid, j1], wsem1)
            return None

        lax.fori_loop(0, N_CHUNKS // 2, body, None)
        pltpu.make_async_copy(rows0, out_hbm.at[wid, N_CHUNKS - 2],
                              wsem0).wait()
        pltpu.make_async_copy(rows1, out_hbm.at[wid, N_CHUNKS - 1],
                              wsem1).wait()

    species_blocked = species.reshape(NW, B_PER_W)
    table_flat = table.reshape(ZMAXPAD * DIM)
    out = sc_gather(species_blocked, table_flat)
    return out.reshape(N_ATOMS, DIM)


# speedup vs baseline: 2.9576x; 2.9576x over previous
import functools

import jax
import jax.numpy as jnp
from jax import lax
from jax.experimental import pallas as pl
from jax.experimental.pallas import tpu as pltpu
from jax.experimental.pallas import tpu_sc as plsc

ZMAXPAD = 88
DIM = 64
N_ATOMS = 1048576

NC = 2
NS = 16
NW = NC * NS
B_PER_W = N_ATOMS // NW
CHUNK = 128
N_CHUNKS = B_PER_W // CHUNK
NBUF = 4


def kernel(species, table):
    mesh = plsc.VectorSubcoreMesh(core_axis_name="c", subcore_axis_name="s")

    @functools.partial(
        pl.kernel,
        mesh=mesh,
        compiler_params=pltpu.CompilerParams(use_tc_tiling_on_sc=False),
        out_type=jax.ShapeDtypeStruct((NW, N_CHUNKS, CHUNK, DIM), jnp.float32),
        scratch_types=[
            pltpu.VMEM((N_CHUNKS, CHUNK), jnp.int32),
            [pltpu.VMEM((CHUNK, DIM), jnp.float32) for _ in range(NBUF)],
            [pltpu.SemaphoreType.DMA for _ in range(NBUF)],
            [pltpu.SemaphoreType.DMA for _ in range(NBUF)],
        ],
    )
    def sc_gather(species_hbm, table_hbm, out_hbm, idx_v, rows, gsems, wsems):
        wid = lax.axis_index("s") * NC + lax.axis_index("c")
        pltpu.sync_copy(species_hbm.at[wid], idx_v)
        my_table = table_hbm.at[wid]

        def body(jj, _):
            base_j = NBUF * jj
            for k in range(NBUF):
                j = base_j + k

                @pl.when(jj > 0)
                def _drain_write():
                    pltpu.make_async_copy(rows[k], out_hbm.at[wid, j],
                                          wsems[k]).wait()

                pltpu.async_copy(my_table.at[idx_v.at[j]], rows[k], gsems[k])

            for k in range(NBUF):
                j = base_j + k
                pltpu.make_async_copy(my_table.at[idx_v.at[j]], rows[k],
                                      gsems[k]).wait()
                pltpu.async_copy(rows[k], out_hbm.at[wid, j], wsems[k])
            return None

        lax.fori_loop(0, N_CHUNKS // NBUF, body, None)
        for k in range(NBUF):
            pltpu.make_async_copy(rows[k], out_hbm.at[wid, N_CHUNKS - NBUF + k],
                                  wsems[k]).wait()

    species_blocked = species.reshape(NW, N_CHUNKS, CHUNK)
    table_rep = jnp.tile(table[None], (NW, 1, 1))
    out = sc_gather(species_blocked, table_rep)
    return out.reshape(N_ATOMS, DIM)
